# baseline (speedup 1.0000x reference)
import jax
import jax.numpy as jnp
from jax import lax
from jax.experimental import pallas as pl

_BH = 16


def _crop_body(rois_ref, data_ref, out_ref):
    bh, w, n = data_ref.shape
    h0 = (pl.program_id(0) * bh).astype(jnp.float32)
    x1 = rois_ref[0, :][None, None, :]
    y1 = rois_ref[1, :][None, None, :]
    x2 = rois_ref[2, :][None, None, :]
    y2 = rois_ref[3, :][None, None, :]
    ww = lax.broadcasted_iota(jnp.int32, (1, w, 1), 1).astype(jnp.float32)
    hh = lax.broadcasted_iota(jnp.int32, (bh, 1, 1), 0).astype(jnp.float32) + h0
    xm = (ww >= x1) & (ww <= x2)
    ym = (hh >= y1) & (hh <= y2)
    inside = xm & ym
    out_ref[...] = jnp.where(inside, data_ref[...], 0.0)


def kernel(data, rois):
    h, w, n = data.shape
    rois_t = rois.T
    grid = (h // _BH,)
    return pl.pallas_call(
        _crop_body,
        grid=grid,
        in_specs=[
            pl.BlockSpec((4, n), lambda i: (0, 0)),
            pl.BlockSpec((_BH, w, n), lambda i: (i, 0, 0)),
        ],
        out_specs=pl.BlockSpec((_BH, w, n), lambda i: (i, 0, 0)),
        out_shape=jax.ShapeDtypeStruct((h, w, n), data.dtype),
    )(rois_t, data)


# speedup vs baseline: 1.0065x; 1.0065x over previous
import jax
import jax.numpy as jnp
from jax import lax
from jax.experimental import pallas as pl
from jax.experimental.pallas import tpu as pltpu

_BH = 32


def _crop_body(rois_ref, data_ref, out_ref):
    bh, w, n = data_ref.shape
    h0 = (pl.program_id(0) * bh).astype(jnp.float32)
    x1 = rois_ref[0, :][None, None, :]
    y1 = rois_ref[1, :][None, None, :]
    x2 = rois_ref[2, :][None, None, :]
    y2 = rois_ref[3, :][None, None, :]
    ww = lax.broadcasted_iota(jnp.int32, (1, w, 1), 1).astype(jnp.float32)
    hh = lax.broadcasted_iota(jnp.int32, (bh, 1, 1), 0).astype(jnp.float32) + h0
    xm = (ww >= x1) & (ww <= x2)
    ym = (hh >= y1) & (hh <= y2)
    inside = xm & ym
    out_ref[...] = jnp.where(inside, data_ref[...], 0.0)


def kernel(data, rois):
    h, w, n = data.shape
    rois_t = rois.T
    grid = (h // _BH,)
    return pl.pallas_call(
        _crop_body,
        grid=grid,
        in_specs=[
            pl.BlockSpec((4, n), lambda i: (0, 0)),
            pl.BlockSpec((_BH, w, n), lambda i: (i, 0, 0)),
        ],
        out_specs=pl.BlockSpec((_BH, w, n), lambda i: (i, 0, 0)),
        out_shape=jax.ShapeDtypeStruct((h, w, n), data.dtype),
        compiler_params=pltpu.CompilerParams(
            dimension_semantics=("parallel",),
        ),
    )(rois_t, data)
